# baseline (speedup 1.0000x reference)
import math

import jax
import jax.numpy as jnp
from jax import lax
from jax.experimental import pallas as pl
from jax.experimental.pallas import tpu as pltpu
from jax.experimental.pallas import tpu_sc as plsc
from jax.experimental import layout as jex_layout

D = 64
NUM_WORKERS = 32
SEQ = 200
BPW = 4096 // NUM_WORKERS
SCALE = math.sqrt(64.0)
LANES = 16
SPLITS = ((0, 128), (128, 72))


def _gather_body(x_hbm, tab_hbm, out_hbm, idx_v, rows_v,
                 isem0, isem1, gsem0, gsem1, osem0, osem1):
    c = lax.axis_index("c")
    s = lax.axis_index("s")
    wid = s * 2 + c
    isems = (isem0, isem1)
    gsems = (gsem0, gsem1)
    osems = (osem0, osem1)

    def fire_stage(j, slot):
        pltpu.async_copy(x_hbm.at[wid * BPW + j], idx_v.at[slot],
                         isems[slot])

    def wait_stage(slot):
        pltpu.make_async_copy(x_hbm.at[wid * BPW], idx_v.at[slot],
                              isems[slot]).wait()

    def fire_gather(slot):
        for off, n in SPLITS:
            pltpu.async_copy(
                tab_hbm.at[idx_v.at[slot].at[pl.ds(off, n)]],
                rows_v.at[slot].at[pl.ds(off, n)],
                gsems[slot],
            )

    def wait_gather(slot):
        for off, n in SPLITS:
            pltpu.make_async_copy(
                tab_hbm.at[idx_v.at[slot].at[pl.ds(off, n)]],
                rows_v.at[slot].at[pl.ds(off, n)],
                gsems[slot],
            ).wait()

    def fire_out(j, slot):
        pltpu.async_copy(
            rows_v.at[slot], out_hbm.at[wid * BPW + j], osems[slot]
        )

    def wait_out(j, slot):
        pltpu.make_async_copy(
            rows_v.at[slot], out_hbm.at[wid * BPW + j], osems[slot]
        ).wait()

    fire_stage(0, 0)
    wait_stage(0)
    fire_gather(0)
    fire_stage(1, 1)

    def body(j2, carry):
        for b in range(2):
            j = j2 * 2 + b
            other = 1 - b

            @pl.when(j2 * 2 + b < BPW - 1)
            def _():
                wait_stage(other)

                if b == 0:
                    @pl.when(j2 >= 1)
                    def _():
                        wait_out(j - 1, other)
                else:
                    wait_out(j - 1, other)

                fire_gather(other)

                @pl.when(j2 * 2 + b < BPW - 2)
                def _():
                    fire_stage(j + 2, b)

            wait_gather(b)
            fire_out(j, b)
        return carry

    lax.fori_loop(0, BPW // 2, body, 0)
    wait_out(BPW - 2, 0)
    wait_out(BPW - 1, 1)


def _retile_body(a_hbm, out_hbm, inv, ov, isem0, isem1, osem0, osem1):
    c = lax.axis_index("c")
    s = lax.axis_index("s")
    wid = s * 2 + c
    isems = (isem0, isem1)
    osems = (osem0, osem1)

    iota = lax.iota(jnp.int32, LANES)
    hi = lax.shift_right_logical(iota, 1 + 2)
    lo = lax.bitwise_and(iota, 7)
    col_base = lo * 128

    def fire_in(seq, slot):
        pltpu.async_copy(a_hbm.at[pl.ds(wid * 128, 128), seq],
                         inv.at[slot], isems[slot])

    def wait_in(slot):
        pltpu.make_async_copy(a_hbm.at[pl.ds(0, 128), 0],
                              inv.at[slot], isems[slot]).wait()

    def fire_out(seq, slot):
        pltpu.async_copy(ov.at[slot], out_hbm.at[seq, pl.ds(0, 8), wid],
                         osems[slot])

    def wait_out(slot):
        pltpu.make_async_copy(ov.at[slot], out_hbm.at[0, pl.ds(0, 8), 0],
                              osems[slot]).wait()

    def transpose(slot):
        def bbody(b, carry):
            for fc in range(D // LANES):
                v = inv[slot, b, pl.ds(fc * LANES, LANES)]
                plsc.store_scatter(
                    ov.at[slot],
                    [hi + 2 * fc, col_base + b],
                    v * SCALE,
                )
            return carry

        lax.fori_loop(0, 128, bbody, 0)

    fire_in(0, 0)
    fire_in(1, 1)

    def body(s2, carry):
        for bslot in range(2):
            seq = s2 * 2 + bslot
            wait_in(bslot)

            @pl.when(seq >= 2)
            def _():
                wait_out(bslot)

            transpose(bslot)
            fire_out(seq, bslot)

            @pl.when(seq < SEQ - 2)
            def _():
                fire_in(seq + 2, bslot)
        return carry

    lax.fori_loop(0, SEQ // 2, body, 0)
    wait_out(0)
    wait_out(1)


@jax.jit
def kernel(x, table):
    rows, cols = x.shape
    table = jex_layout.with_layout_constraint(
        table,
        jex_layout.Layout(major_to_minor=(1, 0), tiling=((8,), (1024,))),
    )
    mesh = plsc.VectorSubcoreMesh(core_axis_name="c", subcore_axis_name="s")

    gathered = pl.kernel(
        _gather_body,
        out_type=jax.ShapeDtypeStruct((rows, cols, D), jnp.float32),
        mesh=mesh,
        compiler_params=pltpu.CompilerParams(use_tc_tiling_on_sc=False),
        scratch_types=[
            pltpu.VMEM((2, SEQ), jnp.int32),
            pltpu.VMEM((2, SEQ, D), jnp.float32),
            pltpu.SemaphoreType.DMA,
            pltpu.SemaphoreType.DMA,
            pltpu.SemaphoreType.DMA,
            pltpu.SemaphoreType.DMA,
            pltpu.SemaphoreType.DMA,
            pltpu.SemaphoreType.DMA,
        ],
    )(x, table)

    tiled = pl.kernel(
        _retile_body,
        out_type=jax.ShapeDtypeStruct((cols, 8, NUM_WORKERS, 1024),
                                      jnp.float32),
        mesh=mesh,
        compiler_params=pltpu.CompilerParams(use_tc_tiling_on_sc=False,
                                             needs_layout_passes=False),
        scratch_types=[
            pltpu.VMEM((2, 128, D), jnp.float32),
            pltpu.VMEM((2, 8, 1024), jnp.float32),
            pltpu.SemaphoreType.DMA,
            pltpu.SemaphoreType.DMA,
            pltpu.SemaphoreType.DMA,
            pltpu.SemaphoreType.DMA,
        ],
    )(gathered)

    out5 = tiled.reshape(cols, 8, NUM_WORKERS, 8, 128)
    out = out5.transpose(2, 4, 0, 1, 3).reshape(rows, cols, D)
    return out


# speedup vs baseline: 2.0001x; 2.0001x over previous
import math

import jax
import jax.numpy as jnp
from jax import lax
from jax.experimental import pallas as pl
from jax.experimental.pallas import tpu as pltpu
from jax.experimental.pallas import tpu_sc as plsc
from jax.experimental import layout as jex_layout

D = 64
NUM_WORKERS = 32
SEQ = 200
BPW = 4096 // NUM_WORKERS
SCALE = math.sqrt(64.0)
LANES = 16
SPLITS = ((0, 128), (128, 72))


def _emb_body(x_hbm, tab_hbm, out_hbm, idx_v, rows_v,
              isem0, isem1, gsem0, gsem1, osem0, osem1):
    c = lax.axis_index("c")
    s = lax.axis_index("s")
    wid = s * 2 + c
    isems = (isem0, isem1)
    gsems = (gsem0, gsem1)
    osems = (osem0, osem1)

    def fire_stage(j, slot):
        pltpu.async_copy(x_hbm.at[wid * BPW + j], idx_v.at[slot],
                         isems[slot])

    def wait_stage(slot):
        pltpu.make_async_copy(x_hbm.at[wid * BPW], idx_v.at[slot],
                              isems[slot]).wait()

    def fire_gather(slot):
        for off, n in SPLITS:
            pltpu.async_copy(
                tab_hbm.at[idx_v.at[slot].at[pl.ds(off, n)]],
                rows_v.at[slot].at[pl.ds(off, n)],
                gsems[slot],
            )

    def wait_gather(slot):
        for off, n in SPLITS:
            pltpu.make_async_copy(
                tab_hbm.at[idx_v.at[slot].at[pl.ds(off, n)]],
                rows_v.at[slot].at[pl.ds(off, n)],
                gsems[slot],
            ).wait()

    def fire_out(j, slot):
        pltpu.async_copy(
            rows_v.at[slot],
            out_hbm.at[wid * BPW + j].at[:, pl.ds(0, D)],
            osems[slot],
        )

    def wait_out(j, slot):
        pltpu.make_async_copy(
            rows_v.at[slot],
            out_hbm.at[wid * BPW + j].at[:, pl.ds(0, D)],
            osems[slot],
        ).wait()

    def scale(slot):
        def group_body(k, carry):
            for i in range(8):
                r = k * 8 + i
                for cc in range(D // LANES):
                    sl = pl.ds(cc * LANES, LANES)
                    rows_v[slot, r, sl] = rows_v[slot, r, sl] * SCALE
            return carry

        lax.fori_loop(0, SEQ // 8, group_body, 0)

    fire_stage(0, 0)
    wait_stage(0)
    fire_gather(0)
    fire_stage(1, 1)

    def body(j2, carry):
        for b in range(2):
            j = j2 * 2 + b
            other = 1 - b

            @pl.when(j2 * 2 + b < BPW - 1)
            def _():
                wait_stage(other)

                if b == 0:
                    @pl.when(j2 >= 1)
                    def _():
                        wait_out(j - 1, other)
                else:
                    wait_out(j - 1, other)

                fire_gather(other)

                @pl.when(j2 * 2 + b < BPW - 2)
                def _():
                    fire_stage(j + 2, b)

            wait_gather(b)
            scale(b)
            fire_out(j, b)
        return carry

    lax.fori_loop(0, BPW // 2, body, 0)
    wait_out(BPW - 2, 0)
    wait_out(BPW - 1, 1)


@jax.jit
def kernel(x, table):
    rows, cols = x.shape
    table = jex_layout.with_layout_constraint(
        table,
        jex_layout.Layout(major_to_minor=(1, 0), tiling=((8,), (1024,))),
    )
    mesh = plsc.VectorSubcoreMesh(core_axis_name="c", subcore_axis_name="s")
    padded = pl.kernel(
        _emb_body,
        out_type=jax.ShapeDtypeStruct((rows, cols, 2 * D), jnp.float32),
        mesh=mesh,
        compiler_params=pltpu.CompilerParams(use_tc_tiling_on_sc=False),
        scratch_types=[
            pltpu.VMEM((2, SEQ), jnp.int32),
            pltpu.VMEM((2, SEQ, D), jnp.float32),
            pltpu.SemaphoreType.DMA,
            pltpu.SemaphoreType.DMA,
            pltpu.SemaphoreType.DMA,
            pltpu.SemaphoreType.DMA,
            pltpu.SemaphoreType.DMA,
            pltpu.SemaphoreType.DMA,
        ],
    )(x, table)
    return padded[:, :, :D]
